# baseline (speedup 1.0000x reference)
import jax
import jax.numpy as jnp
from jax import lax
from jax.experimental import pallas as pl
from jax.experimental.pallas import tpu as pltpu
from jax.experimental.pallas import tpu_sc as plsc

H = 2048
I = 2048
NE = 8
T = 2048

BT = 256
NT = T // BT + NE
TPAD = NT * BT
CJ = 1024
NJ = I // CJ

BTR = 256
BTS = 256
CJS = 1024
NJS = I // CJS


_SC_MESH = plsc.VectorSubcoreMesh(core_axis_name="c", subcore_axis_name="s")
NW = 32
RPT_D = TPAD // NW
RPT_C = T // NW
RC = 32


def _dispatch_body(eidx_hbm, xs_hbm, xsort_hbm, pos_hbm, te_hbm,
                   eidx_v, pos_v, te_v, cnt_s, end_s,
                   pos_sh, idx_v, rows_v, sem):
    cid = lax.axis_index("c")
    sid = lax.axis_index("s")
    lanes = lax.iota(jnp.int32, 16)

    @pl.when(sid == 0)
    def _meta():
        pltpu.sync_copy(eidx_hbm, eidx_v)
        for e in range(NE):
            cnt_s[e] = 0

        def _count(i, c):
            ev = eidx_v[pl.ds(i * 16, 16)]
            for k in range(16):
                e = ev[k]
                cnt_s[e] = cnt_s[e] + 1
            return c
        lax.fori_loop(0, T // 16, _count, 0)

        run = 0
        for e in range(NE):
            pe = ((cnt_s[e] + (BT - 1)) >> 8) << 8
            cnt_s[e] = run
            run = run + pe
            end_s[e] = run

        tb = lanes * BT
        acc = jnp.zeros((16,), jnp.int32)
        for e in range(NE):
            acc = acc + jnp.where(tb >= end_s[e], 1, 0)
        te_v[...] = jnp.minimum(acc, NE - 1)

        def _placepos(i, c):
            ev = eidx_v[pl.ds(i * 16, 16)]
            pv = jnp.zeros((16,), jnp.int32)
            for k in range(16):
                e = ev[k]
                p = cnt_s[e]
                cnt_s[e] = p + 1
                pv = jnp.where(lanes == k, p, pv)
            pos_v[pl.ds(i * 16, 16)] = pv
            return c
        lax.fori_loop(0, T // 16, _placepos, 0)

        pltpu.sync_copy(pos_v, pos_sh)

        @pl.when(cid == 0)
        def _():
            pltpu.sync_copy(pos_v, pos_hbm)
            pltpu.sync_copy(te_v, te_hbm)

    plsc.subcore_barrier()

    wid = cid * 16 + sid
    for k in range(RPT_C // RC):
        base = wid * RPT_C + k * RC
        pltpu.sync_copy(pos_sh.at[pl.ds(base, RC)], idx_v)
        pltpu.sync_copy(xs_hbm.at[pl.ds(base, RC)], rows_v)
        pltpu.async_copy(rows_v, xsort_hbm.at[idx_v], sem).wait()


def _combine_body(pos_hbm, routed_hbm, out_hbm, idx_v, rows_v, sem):
    wid = lax.axis_index("c") * 16 + lax.axis_index("s")
    for k in range(RPT_C // RC):
        base = wid * RPT_C + k * RC
        pltpu.sync_copy(pos_hbm.at[pl.ds(base, RC)], idx_v)
        pltpu.async_copy(routed_hbm.at[idx_v], rows_v, sem).wait()
        pltpu.sync_copy(rows_v, out_hbm.at[pl.ds(base, RC)])


def _router_body(x_ref, wr_ref, logits_ref, eidx_ref, xs_ref):
    x = x_ref[...]
    logits = jnp.dot(x, wr_ref[...], preferred_element_type=jnp.float32)
    m = jnp.max(logits, axis=1, keepdims=True)
    a = jnp.argmax(logits, axis=1).astype(jnp.int32)
    logits_ref[...] = logits
    eidx_ref[...] = a[:, None]
    xs_ref[...] = x * jax.nn.sigmoid(m)


def _gmm_a_body(te_ref, x_ref, g_ref, u_ref, h_ref):
    x = x_ref[...].astype(jnp.bfloat16)
    g = jnp.dot(x, g_ref[0].astype(jnp.bfloat16),
                preferred_element_type=jnp.float32)
    u = jnp.dot(x, u_ref[0].astype(jnp.bfloat16),
                preferred_element_type=jnp.float32)
    h_ref[...] = u * g * jax.nn.sigmoid(g)


def _gmm_b_body(te_ref, h_ref, d_ref, o_ref):
    o_ref[...] = jnp.dot(h_ref[...].astype(jnp.bfloat16),
                         d_ref[0].astype(jnp.bfloat16),
                         preferred_element_type=jnp.float32)


def _shared_a_body(x_ref, gw_ref, uw_ref, s1_ref):
    x = x_ref[...]
    g = jnp.dot(x, gw_ref[...], preferred_element_type=jnp.float32)
    u = jnp.dot(x, uw_ref[...], preferred_element_type=jnp.float32)
    s1_ref[...] = u * g * jax.nn.sigmoid(g)


def _shared_b_body(s1_ref, dw_ref, r_ref, o_ref):
    o_ref[...] = r_ref[...] + jnp.dot(
        s1_ref[...], dw_ref[...], preferred_element_type=jnp.float32)


def kernel(hidden_states, router_weight, gate_up_proj, down_proj,
           shared_gate_w, shared_up_w, shared_down_w):
    x = hidden_states.reshape(T, H)

    logits, eidx2, xs = pl.pallas_call(
        _router_body,
        grid=(T // BTR,),
        in_specs=[pl.BlockSpec((BTR, H), lambda i: (i, 0)),
                  pl.BlockSpec((H, NE), lambda i: (0, 0))],
        out_specs=[pl.BlockSpec((BTR, NE), lambda i: (i, 0)),
                   pl.BlockSpec((BTR, 1), lambda i: (i, 0)),
                   pl.BlockSpec((BTR, H), lambda i: (i, 0))],
        out_shape=[jax.ShapeDtypeStruct((T, NE), jnp.float32),
                   jax.ShapeDtypeStruct((T, 1), jnp.int32),
                   jax.ShapeDtypeStruct((T, H), jnp.float32)],
    )(x, router_weight)
    eidx = eidx2[:, 0]

    x_sorted, pos, te = pl.kernel(
        _dispatch_body, mesh=_SC_MESH,
        out_type=[jax.ShapeDtypeStruct((TPAD, H), jnp.float32),
                  jax.ShapeDtypeStruct((T,), jnp.int32),
                  jax.ShapeDtypeStruct((NT,), jnp.int32)],
        scratch_types=[pltpu.VMEM((T,), jnp.int32),
                       pltpu.VMEM((T,), jnp.int32),
                       pltpu.VMEM((16,), jnp.int32),
                       pltpu.SMEM((16,), jnp.int32),
                       pltpu.SMEM((16,), jnp.int32),
                       pltpu.VMEM_SHARED((T,), jnp.int32),
                       pltpu.VMEM((RC,), jnp.int32),
                       pltpu.VMEM((RC, H), jnp.float32),
                       pltpu.SemaphoreType.DMA],
    )(eidx, xs)

    grid_a = pltpu.PrefetchScalarGridSpec(
        num_scalar_prefetch=1,
        grid=(NJ, NT),
        in_specs=[pl.BlockSpec((BT, H), lambda j, i, te: (i, 0)),
                  pl.BlockSpec((1, H, CJ), lambda j, i, te: (te[i], 0, j)),
                  pl.BlockSpec((1, H, CJ), lambda j, i, te: (te[i], 0, NJ + j))],
        out_specs=pl.BlockSpec((BT, CJ), lambda j, i, te: (i, j)),
    )
    h = pl.pallas_call(
        _gmm_a_body, grid_spec=grid_a,
        out_shape=jax.ShapeDtypeStruct((TPAD, I), jnp.float32),
    )(te, x_sorted, gate_up_proj, gate_up_proj)

    grid_b = pltpu.PrefetchScalarGridSpec(
        num_scalar_prefetch=1,
        grid=(NT,),
        in_specs=[pl.BlockSpec((BT, I), lambda i, te: (i, 0)),
                  pl.BlockSpec((1, I, H), lambda i, te: (te[i], 0, 0))],
        out_specs=pl.BlockSpec((BT, H), lambda i, te: (i, 0)),
    )
    routed_sorted = pl.pallas_call(
        _gmm_b_body, grid_spec=grid_b,
        out_shape=jax.ShapeDtypeStruct((TPAD, H), jnp.float32),
    )(te, h, down_proj)

    routed_tok = pl.kernel(
        _combine_body, mesh=_SC_MESH,
        out_type=jax.ShapeDtypeStruct((T, H), jnp.float32),
        scratch_types=[pltpu.VMEM((RC,), jnp.int32),
                       pltpu.VMEM((RC, H), jnp.float32),
                       pltpu.SemaphoreType.DMA],
    )(pos, routed_sorted)

    s1 = pl.pallas_call(
        _shared_a_body,
        grid=(NJS, T // BTS),
        in_specs=[pl.BlockSpec((BTS, H), lambda j, i: (i, 0)),
                  pl.BlockSpec((H, CJS), lambda j, i: (0, j)),
                  pl.BlockSpec((H, CJS), lambda j, i: (0, j))],
        out_specs=pl.BlockSpec((BTS, CJS), lambda j, i: (i, j)),
        out_shape=jax.ShapeDtypeStruct((T, I), jnp.float32),
    )(x, shared_gate_w, shared_up_w)

    out = pl.pallas_call(
        _shared_b_body,
        grid=(T // BTS,),
        in_specs=[pl.BlockSpec((BTS, I), lambda i: (i, 0)),
                  pl.BlockSpec((I, H), lambda i: (0, 0)),
                  pl.BlockSpec((BTS, H), lambda i: (i, 0))],
        out_specs=pl.BlockSpec((BTS, H), lambda i: (i, 0)),
        out_shape=jax.ShapeDtypeStruct((T, H), jnp.float32),
    )(s1, shared_down_w, routed_tok)

    return out, logits


# speedup vs baseline: 1.0155x; 1.0155x over previous
import jax
import jax.numpy as jnp
from jax import lax
from jax.experimental import pallas as pl
from jax.experimental.pallas import tpu as pltpu
from jax.experimental.pallas import tpu_sc as plsc

H = 2048
I = 2048
NE = 8
T = 2048

BT = 256
NT = T // BT + NE
TPAD = NT * BT
CJ = 1024
NJ = I // CJ

BTR = 256
BTS = 256
CJS = 1024
NJS = I // CJS


_SC_MESH = plsc.VectorSubcoreMesh(core_axis_name="c", subcore_axis_name="s")
NW = 32
RPT_D = TPAD // NW
RPT_C = T // NW
RC = 32


def _dispatch_body(eidx_hbm, xs_hbm, xsort_hbm, pos_hbm, te_hbm, ti_hbm,
                   eidx_v, pos_v, te_v, ti_v, cnt_s, end_s,
                   pos_sh, idx_v, rows_v, sem):
    cid = lax.axis_index("c")
    sid = lax.axis_index("s")
    lanes = lax.iota(jnp.int32, 16)

    @pl.when(sid == 0)
    def _meta():
        pltpu.sync_copy(eidx_hbm, eidx_v)
        for e in range(NE):
            cnt_s[e] = 0

        def _count(i, c):
            ev = eidx_v[pl.ds(i * 16, 16)]
            for k in range(16):
                e = ev[k]
                cnt_s[e] = cnt_s[e] + 1
            return c
        lax.fori_loop(0, T // 16, _count, 0)

        run = 0
        for e in range(NE):
            pe = ((cnt_s[e] + (BT - 1)) >> 8) << 8
            cnt_s[e] = run
            run = run + pe
            end_s[e] = run

        tb = lanes * BT
        acc = jnp.zeros((16,), jnp.int32)
        for e in range(NE):
            acc = acc + jnp.where(tb >= end_s[e], 1, 0)
        used = end_s[NE - 1] >> 8
        last_e = 0
        for e in range(1, NE):
            last_e = jnp.where(end_s[e] > end_s[e - 1], e, last_e)
        is_used = lanes < used
        te_v[...] = jnp.where(is_used, jnp.minimum(acc, NE - 1), last_e)
        ti_v[...] = jnp.where(is_used, lanes, NT - 1)

        def _placepos(i, c):
            ev = eidx_v[pl.ds(i * 16, 16)]
            pv = jnp.zeros((16,), jnp.int32)
            for k in range(16):
                e = ev[k]
                p = cnt_s[e]
                cnt_s[e] = p + 1
                pv = jnp.where(lanes == k, p, pv)
            pos_v[pl.ds(i * 16, 16)] = pv
            return c
        lax.fori_loop(0, T // 16, _placepos, 0)

        pltpu.sync_copy(pos_v, pos_sh)

        @pl.when(cid == 0)
        def _():
            pltpu.sync_copy(pos_v, pos_hbm)
            pltpu.sync_copy(te_v, te_hbm)
            pltpu.sync_copy(ti_v, ti_hbm)

    plsc.subcore_barrier()

    wid = cid * 16 + sid
    for k in range(RPT_C // RC):
        base = wid * RPT_C + k * RC
        pltpu.sync_copy(pos_sh.at[pl.ds(base, RC)], idx_v)
        pltpu.sync_copy(xs_hbm.at[pl.ds(base, RC)], rows_v)
        pltpu.async_copy(rows_v, xsort_hbm.at[idx_v], sem).wait()


def _combine_body(pos_hbm, routed_hbm, out_hbm, idx_v, rows_v, sem):
    wid = lax.axis_index("c") * 16 + lax.axis_index("s")
    for k in range(RPT_C // RC):
        base = wid * RPT_C + k * RC
        pltpu.sync_copy(pos_hbm.at[pl.ds(base, RC)], idx_v)
        pltpu.async_copy(routed_hbm.at[idx_v], rows_v, sem).wait()
        pltpu.sync_copy(rows_v, out_hbm.at[pl.ds(base, RC)])


def _router_body(x_ref, wr_ref, logits_ref, eidx_ref, xs_ref):
    x = x_ref[...]
    logits = jnp.dot(x, wr_ref[...], preferred_element_type=jnp.float32)
    m = jnp.max(logits, axis=1, keepdims=True)
    a = jnp.argmax(logits, axis=1).astype(jnp.int32)
    logits_ref[...] = logits
    eidx_ref[...] = a[:, None]
    xs_ref[...] = x * jax.nn.sigmoid(m)


def _gmm_a_body(te_ref, ti_ref, x_ref, g_ref, u_ref, h_ref):
    x = x_ref[...]
    g = jnp.dot(x, g_ref[0], preferred_element_type=jnp.float32)
    u = jnp.dot(x, u_ref[0], preferred_element_type=jnp.float32)
    h_ref[...] = (u * g * jax.nn.sigmoid(g)).astype(jnp.bfloat16)


def _gmm_b_body(te_ref, ti_ref, h_ref, d_ref, o_ref):
    o_ref[...] = jnp.dot(h_ref[...], d_ref[0].astype(jnp.bfloat16),
                         preferred_element_type=jnp.float32)


def _shared_a_body(x_ref, gw_ref, uw_ref, s1_ref):
    x = x_ref[...]
    g = jnp.dot(x, gw_ref[...], preferred_element_type=jnp.float32)
    u = jnp.dot(x, uw_ref[...], preferred_element_type=jnp.float32)
    s1_ref[...] = (u * g * jax.nn.sigmoid(g)).astype(jnp.bfloat16)


def _shared_b_body(s1_ref, dw_ref, r_ref, o_ref):
    o_ref[...] = r_ref[...] + jnp.dot(
        s1_ref[...], dw_ref[...].astype(jnp.bfloat16),
        preferred_element_type=jnp.float32)


def kernel(hidden_states, router_weight, gate_up_proj, down_proj,
           shared_gate_w, shared_up_w, shared_down_w):
    x = hidden_states.reshape(T, H)

    logits, eidx2, xs = pl.pallas_call(
        _router_body,
        grid=(T // BTR,),
        in_specs=[pl.BlockSpec((BTR, H), lambda i: (i, 0)),
                  pl.BlockSpec((H, NE), lambda i: (0, 0))],
        out_specs=[pl.BlockSpec((BTR, NE), lambda i: (i, 0)),
                   pl.BlockSpec((BTR, 1), lambda i: (i, 0)),
                   pl.BlockSpec((BTR, H), lambda i: (i, 0))],
        out_shape=[jax.ShapeDtypeStruct((T, NE), jnp.float32),
                   jax.ShapeDtypeStruct((T, 1), jnp.int32),
                   jax.ShapeDtypeStruct((T, H), jnp.float32)],
    )(x, router_weight)
    eidx = eidx2[:, 0]

    x_sorted, pos, te, ti = pl.kernel(
        _dispatch_body, mesh=_SC_MESH,
        out_type=[jax.ShapeDtypeStruct((TPAD, H), jnp.float32),
                  jax.ShapeDtypeStruct((T,), jnp.int32),
                  jax.ShapeDtypeStruct((NT,), jnp.int32),
                  jax.ShapeDtypeStruct((NT,), jnp.int32)],
        scratch_types=[pltpu.VMEM((T,), jnp.int32),
                       pltpu.VMEM((T,), jnp.int32),
                       pltpu.VMEM((16,), jnp.int32),
                       pltpu.VMEM((16,), jnp.int32),
                       pltpu.SMEM((16,), jnp.int32),
                       pltpu.SMEM((16,), jnp.int32),
                       pltpu.VMEM_SHARED((T,), jnp.int32),
                       pltpu.VMEM((RC,), jnp.int32),
                       pltpu.VMEM((RC, H), jnp.float32),
                       pltpu.SemaphoreType.DMA],
    )(eidx, xs)

    grid_a = pltpu.PrefetchScalarGridSpec(
        num_scalar_prefetch=2,
        grid=(NJ, NT),
        in_specs=[pl.BlockSpec((BT, H), lambda j, i, te, ti: (ti[i], 0)),
                  pl.BlockSpec((1, H, CJ), lambda j, i, te, ti: (te[i], 0, j)),
                  pl.BlockSpec((1, H, CJ),
                               lambda j, i, te, ti: (te[i], 0, NJ + j))],
        out_specs=pl.BlockSpec((BT, CJ), lambda j, i, te, ti: (ti[i], j)),
    )
    h = pl.pallas_call(
        _gmm_a_body, grid_spec=grid_a,
        out_shape=jax.ShapeDtypeStruct((TPAD, I), jnp.bfloat16),
    )(te, ti, x_sorted, gate_up_proj, gate_up_proj)

    grid_b = pltpu.PrefetchScalarGridSpec(
        num_scalar_prefetch=2,
        grid=(NT,),
        in_specs=[pl.BlockSpec((BT, I), lambda i, te, ti: (ti[i], 0)),
                  pl.BlockSpec((1, I, H), lambda i, te, ti: (te[i], 0, 0))],
        out_specs=pl.BlockSpec((BT, H), lambda i, te, ti: (ti[i], 0)),
    )
    routed_sorted = pl.pallas_call(
        _gmm_b_body, grid_spec=grid_b,
        out_shape=jax.ShapeDtypeStruct((TPAD, H), jnp.float32),
    )(te, ti, h, down_proj)

    routed_tok = pl.kernel(
        _combine_body, mesh=_SC_MESH,
        out_type=jax.ShapeDtypeStruct((T, H), jnp.float32),
        scratch_types=[pltpu.VMEM((RC,), jnp.int32),
                       pltpu.VMEM((RC, H), jnp.float32),
                       pltpu.SemaphoreType.DMA],
    )(pos, routed_sorted)

    s1 = pl.pallas_call(
        _shared_a_body,
        grid=(NJS, T // BTS),
        in_specs=[pl.BlockSpec((BTS, H), lambda j, i: (i, 0)),
                  pl.BlockSpec((H, CJS), lambda j, i: (0, j)),
                  pl.BlockSpec((H, CJS), lambda j, i: (0, j))],
        out_specs=pl.BlockSpec((BTS, CJS), lambda j, i: (i, j)),
        out_shape=jax.ShapeDtypeStruct((T, I), jnp.bfloat16),
    )(x, shared_gate_w, shared_up_w)

    out = pl.pallas_call(
        _shared_b_body,
        grid=(T // BTS,),
        in_specs=[pl.BlockSpec((BTS, I), lambda i: (i, 0)),
                  pl.BlockSpec((I, H), lambda i: (0, 0)),
                  pl.BlockSpec((BTS, H), lambda i: (i, 0))],
        out_specs=pl.BlockSpec((BTS, H), lambda i: (i, 0)),
        out_shape=jax.ShapeDtypeStruct((T, H), jnp.float32),
    )(s1, shared_down_w, routed_tok)

    return out, logits


# speedup vs baseline: 1.0277x; 1.0119x over previous
import jax
import jax.numpy as jnp
from jax import lax
from jax.experimental import pallas as pl
from jax.experimental.pallas import tpu as pltpu
from jax.experimental.pallas import tpu_sc as plsc

H = 2048
I = 2048
NE = 8
T = 2048

BT = 256
NT = T // BT + NE
TPAD = NT * BT
CJ = 1024
NJ = I // CJ

BTR = 256
BTS = 512
CJS = 1024
NJS = I // CJS


_SC_MESH = plsc.VectorSubcoreMesh(core_axis_name="c", subcore_axis_name="s")
NW = 32
RPT_D = TPAD // NW
RPT_C = T // NW
RC = 32


def _dispatch_body(eidx_hbm, xs_hbm, xsort_hbm, pos_hbm, te_hbm, ti_hbm,
                   eidx_v, pos_v, te_v, ti_v, cnt_s, end_s,
                   pos_sh, idx_v, rows_v, sem):
    cid = lax.axis_index("c")
    sid = lax.axis_index("s")
    lanes = lax.iota(jnp.int32, 16)

    @pl.when(sid == 0)
    def _meta():
        pltpu.sync_copy(eidx_hbm, eidx_v)
        for e in range(NE):
            cnt_s[e] = 0

        def _count(i, c):
            ev = eidx_v[pl.ds(i * 16, 16)]
            for k in range(16):
                e = ev[k]
                cnt_s[e] = cnt_s[e] + 1
            return c
        lax.fori_loop(0, T // 16, _count, 0)

        run = 0
        for e in range(NE):
            pe = ((cnt_s[e] + (BT - 1)) >> 8) << 8
            cnt_s[e] = run
            run = run + pe
            end_s[e] = run

        tb = lanes * BT
        acc = jnp.zeros((16,), jnp.int32)
        for e in range(NE):
            acc = acc + jnp.where(tb >= end_s[e], 1, 0)
        used = end_s[NE - 1] >> 8
        last_e = 0
        for e in range(1, NE):
            last_e = jnp.where(end_s[e] > end_s[e - 1], e, last_e)
        is_used = lanes < used
        te_v[...] = jnp.where(is_used, jnp.minimum(acc, NE - 1), last_e)
        ti_v[...] = jnp.where(is_used, lanes, NT - 1)

        def _placepos(i, c):
            ev = eidx_v[pl.ds(i * 16, 16)]
            pv = jnp.zeros((16,), jnp.int32)
            for k in range(16):
                e = ev[k]
                p = cnt_s[e]
                cnt_s[e] = p + 1
                pv = jnp.where(lanes == k, p, pv)
            pos_v[pl.ds(i * 16, 16)] = pv
            return c
        lax.fori_loop(0, T // 16, _placepos, 0)

        pltpu.sync_copy(pos_v, pos_sh)

        @pl.when(cid == 0)
        def _():
            pltpu.sync_copy(pos_v, pos_hbm)
            pltpu.sync_copy(te_v, te_hbm)
            pltpu.sync_copy(ti_v, ti_hbm)

    plsc.subcore_barrier()

    wid = cid * 16 + sid
    for k in range(RPT_C // RC):
        base = wid * RPT_C + k * RC
        pltpu.sync_copy(pos_sh.at[pl.ds(base, RC)], idx_v)
        pltpu.sync_copy(xs_hbm.at[pl.ds(base, RC)], rows_v)
        pltpu.async_copy(rows_v, xsort_hbm.at[idx_v], sem).wait()


def _combine_body(pos_hbm, routed_hbm, out_hbm, idx_v, rows_v, sem):
    wid = lax.axis_index("c") * 16 + lax.axis_index("s")
    for k in range(RPT_C // RC):
        base = wid * RPT_C + k * RC
        pltpu.sync_copy(pos_hbm.at[pl.ds(base, RC)], idx_v)
        pltpu.async_copy(routed_hbm.at[idx_v], rows_v, sem).wait()
        pltpu.sync_copy(rows_v, out_hbm.at[pl.ds(base, RC)])


def _router_body(x_ref, wr_ref, logits_ref, eidx_ref, xs_ref):
    x = x_ref[...]
    logits = jnp.dot(x, wr_ref[...], preferred_element_type=jnp.float32)
    m = jnp.max(logits, axis=1, keepdims=True)
    a = jnp.argmax(logits, axis=1).astype(jnp.int32)
    logits_ref[...] = logits
    eidx_ref[...] = a[:, None]
    xs_ref[...] = x * jax.nn.sigmoid(m)


def _gmm_a_body(te_ref, ti_ref, x_ref, g_ref, u_ref, h_ref):
    x = x_ref[...]
    g = jnp.dot(x, g_ref[0], preferred_element_type=jnp.float32)
    u = jnp.dot(x, u_ref[0], preferred_element_type=jnp.float32)
    h_ref[...] = (u * g * jax.nn.sigmoid(g)).astype(jnp.bfloat16)


def _gmm_b_body(te_ref, ti_ref, h_ref, d_ref, o_ref):
    o_ref[...] = jnp.dot(h_ref[...], d_ref[0].astype(jnp.bfloat16),
                         preferred_element_type=jnp.float32)


def _shared_a_body(x_ref, gw_ref, uw_ref, s1_ref):
    x = x_ref[...]
    g = jnp.dot(x, gw_ref[...], preferred_element_type=jnp.float32)
    u = jnp.dot(x, uw_ref[...], preferred_element_type=jnp.float32)
    s1_ref[...] = (u * g * jax.nn.sigmoid(g)).astype(jnp.bfloat16)


def _shared_b_body(s1_ref, dw_ref, r_ref, o_ref):
    o_ref[...] = r_ref[...] + jnp.dot(
        s1_ref[...], dw_ref[...].astype(jnp.bfloat16),
        preferred_element_type=jnp.float32)


def kernel(hidden_states, router_weight, gate_up_proj, down_proj,
           shared_gate_w, shared_up_w, shared_down_w):
    x = hidden_states.reshape(T, H)

    logits, eidx2, xs = pl.pallas_call(
        _router_body,
        grid=(T // BTR,),
        in_specs=[pl.BlockSpec((BTR, H), lambda i: (i, 0)),
                  pl.BlockSpec((H, NE), lambda i: (0, 0))],
        out_specs=[pl.BlockSpec((BTR, NE), lambda i: (i, 0)),
                   pl.BlockSpec((BTR, 1), lambda i: (i, 0)),
                   pl.BlockSpec((BTR, H), lambda i: (i, 0))],
        out_shape=[jax.ShapeDtypeStruct((T, NE), jnp.float32),
                   jax.ShapeDtypeStruct((T, 1), jnp.int32),
                   jax.ShapeDtypeStruct((T, H), jnp.float32)],
    )(x, router_weight)
    eidx = eidx2[:, 0]

    x_sorted, pos, te, ti = pl.kernel(
        _dispatch_body, mesh=_SC_MESH,
        out_type=[jax.ShapeDtypeStruct((TPAD, H), jnp.float32),
                  jax.ShapeDtypeStruct((T,), jnp.int32),
                  jax.ShapeDtypeStruct((NT,), jnp.int32),
                  jax.ShapeDtypeStruct((NT,), jnp.int32)],
        scratch_types=[pltpu.VMEM((T,), jnp.int32),
                       pltpu.VMEM((T,), jnp.int32),
                       pltpu.VMEM((16,), jnp.int32),
                       pltpu.VMEM((16,), jnp.int32),
                       pltpu.SMEM((16,), jnp.int32),
                       pltpu.SMEM((16,), jnp.int32),
                       pltpu.VMEM_SHARED((T,), jnp.int32),
                       pltpu.VMEM((RC,), jnp.int32),
                       pltpu.VMEM((RC, H), jnp.float32),
                       pltpu.SemaphoreType.DMA],
    )(eidx, xs)

    grid_a = pltpu.PrefetchScalarGridSpec(
        num_scalar_prefetch=2,
        grid=(NJ, NT),
        in_specs=[pl.BlockSpec((BT, H), lambda j, i, te, ti: (ti[i], 0)),
                  pl.BlockSpec((1, H, CJ), lambda j, i, te, ti: (te[i], 0, j)),
                  pl.BlockSpec((1, H, CJ),
                               lambda j, i, te, ti: (te[i], 0, NJ + j))],
        out_specs=pl.BlockSpec((BT, CJ), lambda j, i, te, ti: (ti[i], j)),
    )
    h = pl.pallas_call(
        _gmm_a_body, grid_spec=grid_a,
        out_shape=jax.ShapeDtypeStruct((TPAD, I), jnp.bfloat16),
    )(te, ti, x_sorted, gate_up_proj, gate_up_proj)

    grid_b = pltpu.PrefetchScalarGridSpec(
        num_scalar_prefetch=2,
        grid=(NT,),
        in_specs=[pl.BlockSpec((BT, I), lambda i, te, ti: (ti[i], 0)),
                  pl.BlockSpec((1, I, H), lambda i, te, ti: (te[i], 0, 0))],
        out_specs=pl.BlockSpec((BT, H), lambda i, te, ti: (ti[i], 0)),
    )
    routed_sorted = pl.pallas_call(
        _gmm_b_body, grid_spec=grid_b,
        out_shape=jax.ShapeDtypeStruct((TPAD, H), jnp.float32),
    )(te, ti, h, down_proj)

    routed_tok = pl.kernel(
        _combine_body, mesh=_SC_MESH,
        out_type=jax.ShapeDtypeStruct((T, H), jnp.float32),
        scratch_types=[pltpu.VMEM((RC,), jnp.int32),
                       pltpu.VMEM((RC, H), jnp.float32),
                       pltpu.SemaphoreType.DMA],
    )(pos, routed_sorted)

    s1 = pl.pallas_call(
        _shared_a_body,
        grid=(NJS, T // BTS),
        in_specs=[pl.BlockSpec((BTS, H), lambda j, i: (i, 0)),
                  pl.BlockSpec((H, CJS), lambda j, i: (0, j)),
                  pl.BlockSpec((H, CJS), lambda j, i: (0, j))],
        out_specs=pl.BlockSpec((BTS, CJS), lambda j, i: (i, j)),
        out_shape=jax.ShapeDtypeStruct((T, I), jnp.bfloat16),
    )(x, shared_gate_w, shared_up_w)

    out = pl.pallas_call(
        _shared_b_body,
        grid=(T // BTS,),
        in_specs=[pl.BlockSpec((BTS, I), lambda i: (i, 0)),
                  pl.BlockSpec((I, H), lambda i: (0, 0)),
                  pl.BlockSpec((BTS, H), lambda i: (i, 0))],
        out_specs=pl.BlockSpec((BTS, H), lambda i: (i, 0)),
        out_shape=jax.ShapeDtypeStruct((T, H), jnp.float32),
    )(s1, shared_down_w, routed_tok)

    return out, logits
